# baseline (speedup 1.0000x reference)
import jax
import jax.numpy as jnp
from jax.experimental import pallas as pl
from jax.experimental.pallas import tpu as pltpu


def _recurrent_kernel(emb_ref, enc_ref, mask_ref, h0_ref, c0_ref, wa_ref,
                      w0_ref, b0_ref, w1_ref, b1_ref,
                      hc_ref, attw_ref, h_s, c_s):
    B, T, E = emb_ref.shape
    _, S, I = enc_ref.shape
    H = h0_ref.shape[2]
    H2, H3, H4 = 2 * H, 3 * H, 4 * H

    enc = enc_ref[...]
    kpre = jnp.dot(enc.reshape(B * S, I), wa_ref[...],
                   preferred_element_type=jnp.float32).reshape(B, S, H)

    h_s[...] = h0_ref[...]
    c_s[...] = c0_ref[...]

    w0 = w0_ref[...]
    w1 = w1_ref[...]
    b0 = jnp.broadcast_to(b0_ref[...], (B, H4))
    b1 = jnp.broadcast_to(b1_ref[...], (B, H4))
    mask_bias = jnp.where(mask_ref[...] > 0.0, 0.0, -1e30)

    @pl.loop(0, T)
    def _step(t):
        h_prev0 = h_s[0]
        h_prev1 = h_s[1]

        scores = jnp.sum(h_prev1[:, None, :] * kpre, axis=-1) + mask_bias
        m = jnp.max(scores, axis=-1, keepdims=True)
        p = jnp.exp(scores - m)
        w = p / jnp.sum(p, axis=-1, keepdims=True)
        ctx = jnp.sum(w[:, :, None] * enc, axis=1)

        emb = emb_ref[:, pl.ds(t, 1), :][:, 0, :]

        x0 = jnp.concatenate([emb, ctx, h_prev0], axis=-1)
        g0 = jnp.dot(x0, w0, preferred_element_type=jnp.float32) + b0
        sg0 = jax.nn.sigmoid(g0[:, :H3])
        th0 = jnp.tanh(g0[:, H3:])
        c0n = sg0[:, H:H2] * c_s[0] + sg0[:, :H] * th0
        h0n = sg0[:, H2:H3] * jnp.tanh(c0n)

        x1 = jnp.concatenate([h0n, h_prev1], axis=-1)
        g1 = jnp.dot(x1, w1, preferred_element_type=jnp.float32) + b1
        sg1 = jax.nn.sigmoid(g1[:, :H3])
        th1 = jnp.tanh(g1[:, H3:])
        c1n = sg1[:, H:H2] * c_s[1] + sg1[:, :H] * th1
        h1n = sg1[:, H2:H3] * jnp.tanh(c1n)

        h_s[0] = h0n
        c_s[0] = c0n
        h_s[1] = h1n
        c_s[1] = c1n

        rec = jnp.concatenate([h1n, ctx], axis=-1)
        hc_ref[:, pl.ds(t, 1), :] = rec[:, None, :]
        attw_ref[:, pl.ds(t, 1), :] = w[:, None, :]


def _projection_kernel(hc_ref, emb_ref, wo_ref, bo_ref, ws_ref, bs_ref,
                       out_ref):
    x = jnp.concatenate([hc_ref[...], emb_ref[...]],
                        axis=-1).astype(jnp.bfloat16)
    lin = (jnp.dot(x, wo_ref[...], preferred_element_type=jnp.float32)
           + bo_ref[...])
    act = jnp.tanh(lin).astype(jnp.bfloat16)
    out_ref[...] = (jnp.dot(act, ws_ref[...],
                            preferred_element_type=jnp.float32) + bs_ref[...])


def kernel(embedding, wa_t, w0, b0, w1, b1, wo, bo, ws, bs,
           y_tokens, enc_output, mask, h0, c0):
    B, S, I = enc_output.shape
    L, _, H = h0.shape
    V, E = embedding.shape
    inter = bo.shape[-1]
    T = y_tokens.shape[1] - 1
    H2, H3, H4 = 2 * H, 3 * H, 4 * H
    BH = B // 2

    emb_bte = embedding[y_tokens[:, :T]]
    wa = wa_t.T

    def _perm_gates(m):
        return jnp.concatenate([m[:, :H2], m[:, H3:], m[:, H2:H3]], axis=-1)

    w0p, b0p = _perm_gates(w0), _perm_gates(b0)
    w1p, b1p = _perm_gates(w1), _perm_gates(b1)

    hc, attw = pl.pallas_call(
        _recurrent_kernel,
        out_shape=(jax.ShapeDtypeStruct((B, T, H2), jnp.float32),
                   jax.ShapeDtypeStruct((B, T, S), jnp.float32)),
        grid=(2,),
        in_specs=[
            pl.BlockSpec((BH, T, E), lambda i: (i, 0, 0)),
            pl.BlockSpec((BH, S, I), lambda i: (i, 0, 0)),
            pl.BlockSpec((BH, S), lambda i: (i, 0)),
            pl.BlockSpec((L, BH, H), lambda i: (0, i, 0)),
            pl.BlockSpec((L, BH, H), lambda i: (0, i, 0)),
            pl.BlockSpec((I, H), lambda i: (0, 0)),
            pl.BlockSpec((E + I + H, H4), lambda i: (0, 0)),
            pl.BlockSpec((1, H4), lambda i: (0, 0)),
            pl.BlockSpec((H2, H4), lambda i: (0, 0)),
            pl.BlockSpec((1, H4), lambda i: (0, 0)),
        ],
        out_specs=(pl.BlockSpec((BH, T, H2), lambda i: (i, 0, 0)),
                   pl.BlockSpec((BH, T, S), lambda i: (i, 0, 0))),
        scratch_shapes=[pltpu.VMEM((L, BH, H), jnp.float32),
                        pltpu.VMEM((L, BH, H), jnp.float32)],
        compiler_params=pltpu.CompilerParams(
            dimension_semantics=("parallel",)),
    )(emb_bte, enc_output, mask, h0, c0, wa, w0p, b0p, w1p, b1p)

    rows = B * T
    n_tiles = 8
    rb = rows // n_tiles
    x_hc = hc.reshape(rows, H2)
    x_emb = emb_bte.reshape(rows, E)
    wo_bf = wo.astype(jnp.bfloat16)
    ws_bf = ws.astype(jnp.bfloat16)

    logits = pl.pallas_call(
        _projection_kernel,
        out_shape=jax.ShapeDtypeStruct((rows, V), jnp.float32),
        grid=(n_tiles,),
        in_specs=[
            pl.BlockSpec((rb, H2), lambda i: (i, 0)),
            pl.BlockSpec((rb, E), lambda i: (i, 0)),
            pl.BlockSpec((H + I + E, inter), lambda i: (0, 0)),
            pl.BlockSpec((1, inter), lambda i: (0, 0)),
            pl.BlockSpec((inter, V), lambda i: (0, 0)),
            pl.BlockSpec((1, V), lambda i: (0, 0)),
        ],
        out_specs=pl.BlockSpec((rb, V), lambda i: (i, 0)),
        compiler_params=pltpu.CompilerParams(
            dimension_semantics=("parallel",)),
    )(x_hc, x_emb, wo_bf, bo, ws_bf, bs)

    return logits.reshape(B, T, V), attw


# speedup vs baseline: 1.1362x; 1.1362x over previous
import jax
import jax.numpy as jnp
from jax.experimental import pallas as pl
from jax.experimental.pallas import tpu as pltpu


def _recurrent_kernel(emb_ref, enc_ref, mask_ref, h0_ref, c0_ref, wa_ref,
                      w0_ref, b0_ref, w1_ref, b1_ref,
                      hc_ref, attw_ref, h_s, c_s):
    B, T, E = emb_ref.shape
    _, S, I = enc_ref.shape
    H = h0_ref.shape[2]
    H2, H3, H4 = 2 * H, 3 * H, 4 * H

    enc = enc_ref[...]
    kpre = jnp.dot(enc.reshape(B * S, I), wa_ref[...],
                   preferred_element_type=jnp.float32).reshape(B, S, H)

    h_s[...] = h0_ref[...]
    c_s[...] = c0_ref[...]

    w0 = w0_ref[...]
    w1 = w1_ref[...]
    b0 = jnp.broadcast_to(b0_ref[...], (B, H4))
    b1 = jnp.broadcast_to(b1_ref[...], (B, H4))
    mask_bias = jnp.where(mask_ref[...] > 0.0, 0.0, -1e30)

    @pl.loop(0, T)
    def _step(t):
        h_prev0 = h_s[0]
        h_prev1 = h_s[1]

        scores = jnp.sum(h_prev1[:, None, :] * kpre, axis=-1) + mask_bias
        m = jnp.max(scores, axis=-1, keepdims=True)
        p = jnp.exp(scores - m)
        w = p / jnp.sum(p, axis=-1, keepdims=True)
        ctx = jnp.sum(w[:, :, None] * enc, axis=1)

        emb = emb_ref[:, pl.ds(t, 1), :][:, 0, :]

        x0 = jnp.concatenate([emb, ctx, h_prev0], axis=-1)
        g0 = jnp.dot(x0, w0, preferred_element_type=jnp.float32) + b0
        sg0 = jax.nn.sigmoid(g0[:, :H3])
        th0 = jnp.tanh(g0[:, H3:])
        c0n = sg0[:, H:H2] * c_s[0] + sg0[:, :H] * th0
        h0n = sg0[:, H2:H3] * jnp.tanh(c0n)

        x1 = jnp.concatenate([h0n, h_prev1], axis=-1)
        g1 = jnp.dot(x1, w1, preferred_element_type=jnp.float32) + b1
        sg1 = jax.nn.sigmoid(g1[:, :H3])
        th1 = jnp.tanh(g1[:, H3:])
        c1n = sg1[:, H:H2] * c_s[1] + sg1[:, :H] * th1
        h1n = sg1[:, H2:H3] * jnp.tanh(c1n)

        h_s[0] = h0n
        c_s[0] = c0n
        h_s[1] = h1n
        c_s[1] = c1n

        rec = jnp.concatenate([h1n, ctx], axis=-1)
        hc_ref[:, pl.ds(t, 1), :] = rec[:, None, :]
        attw_ref[:, pl.ds(t, 1), :] = w[:, None, :]


def _projection_kernel(hc_ref, emb_ref, wo_ref, bo_ref, ws_ref, bs_ref,
                       out_ref):
    x = jnp.concatenate([hc_ref[...], emb_ref[...]],
                        axis=-1).astype(jnp.bfloat16)
    lin = (jnp.dot(x, wo_ref[...], preferred_element_type=jnp.float32)
           + bo_ref[...])
    act = jnp.tanh(lin).astype(jnp.bfloat16)
    out_ref[...] = (jnp.dot(act, ws_ref[...],
                            preferred_element_type=jnp.float32) + bs_ref[...])


def kernel(embedding, wa_t, w0, b0, w1, b1, wo, bo, ws, bs,
           y_tokens, enc_output, mask, h0, c0):
    B, S, I = enc_output.shape
    L, _, H = h0.shape
    V, E = embedding.shape
    inter = bo.shape[-1]
    T = y_tokens.shape[1] - 1
    H2, H3, H4 = 2 * H, 3 * H, 4 * H

    emb_bte = embedding[y_tokens[:, :T]]
    wa = wa_t.T

    def _perm_gates(m):
        return jnp.concatenate([m[:, :H2], m[:, H3:], m[:, H2:H3]], axis=-1)

    w0p, b0p = _perm_gates(w0), _perm_gates(b0)
    w1p, b1p = _perm_gates(w1), _perm_gates(b1)

    hc, attw = pl.pallas_call(
        _recurrent_kernel,
        out_shape=(jax.ShapeDtypeStruct((B, T, H2), jnp.float32),
                   jax.ShapeDtypeStruct((B, T, S), jnp.float32)),
        grid=(1,),
        in_specs=[
            pl.BlockSpec((B, T, E), lambda i: (i, 0, 0)),
            pl.BlockSpec((B, S, I), lambda i: (i, 0, 0)),
            pl.BlockSpec((B, S), lambda i: (i, 0)),
            pl.BlockSpec((L, B, H), lambda i: (0, i, 0)),
            pl.BlockSpec((L, B, H), lambda i: (0, i, 0)),
            pl.BlockSpec((I, H), lambda i: (0, 0)),
            pl.BlockSpec((E + I + H, H4), lambda i: (0, 0)),
            pl.BlockSpec((1, H4), lambda i: (0, 0)),
            pl.BlockSpec((H2, H4), lambda i: (0, 0)),
            pl.BlockSpec((1, H4), lambda i: (0, 0)),
        ],
        out_specs=(pl.BlockSpec((B, T, H2), lambda i: (i, 0, 0)),
                   pl.BlockSpec((B, T, S), lambda i: (i, 0, 0))),
        scratch_shapes=[pltpu.VMEM((L, B, H), jnp.float32),
                        pltpu.VMEM((L, B, H), jnp.float32)],
        compiler_params=pltpu.CompilerParams(
            dimension_semantics=("arbitrary",)),
    )(emb_bte, enc_output, mask, h0, c0, wa, w0p, b0p, w1p, b1p)

    rows = B * T
    n_tiles = 8
    rb = rows // n_tiles
    x_hc = hc.reshape(rows, H2)
    x_emb = emb_bte.reshape(rows, E)
    wo_bf = wo.astype(jnp.bfloat16)
    ws_bf = ws.astype(jnp.bfloat16)

    logits = pl.pallas_call(
        _projection_kernel,
        out_shape=jax.ShapeDtypeStruct((rows, V), jnp.float32),
        grid=(n_tiles,),
        in_specs=[
            pl.BlockSpec((rb, H2), lambda i: (i, 0)),
            pl.BlockSpec((rb, E), lambda i: (i, 0)),
            pl.BlockSpec((H + I + E, inter), lambda i: (0, 0)),
            pl.BlockSpec((1, inter), lambda i: (0, 0)),
            pl.BlockSpec((inter, V), lambda i: (0, 0)),
            pl.BlockSpec((1, V), lambda i: (0, 0)),
        ],
        out_specs=pl.BlockSpec((rb, V), lambda i: (i, 0)),
        compiler_params=pltpu.CompilerParams(
            dimension_semantics=("arbitrary",)),
    )(x_hc, x_emb, wo_bf, bo, ws_bf, bs)

    return logits.reshape(B, T, V), attw


# speedup vs baseline: 1.6957x; 1.4925x over previous
import jax
import jax.numpy as jnp
from jax.experimental import pallas as pl
from jax.experimental.pallas import tpu as pltpu


def _recurrent_kernel(emb_ref, enc_ref, mask_ref, h0_ref, c0_ref, wa_ref,
                      w0_ref, b0_ref, w1_ref, b1_ref,
                      hc_ref, attw_ref, h_s, c_s):
    B, T, E = emb_ref.shape
    _, S, I = enc_ref.shape
    H = h0_ref.shape[2]
    H2, H3, H4 = 2 * H, 3 * H, 4 * H
    BS = B * S

    enc = enc_ref[...]
    enc2d = enc.reshape(BS, I)
    kpt = jax.lax.dot_general(wa_ref[...], enc2d,
                              (((0,), (1,)), ((), ())),
                              preferred_element_type=jnp.float32)

    lane_b = jax.lax.broadcasted_iota(jnp.int32, (B, BS), 1) // S
    row_b = jax.lax.broadcasted_iota(jnp.int32, (B, BS), 0)
    diag01 = jnp.where(lane_b == row_b, 1.0, 0.0)
    mask_bias = jnp.where(mask_ref[...] > 0.0, 0.0, -1e30)

    h_s[...] = h0_ref[...]
    c_s[...] = c0_ref[...]

    w0 = w0_ref[...]
    w1 = w1_ref[...]
    b0 = jnp.broadcast_to(b0_ref[...], (B, H4))
    b1 = jnp.broadcast_to(b1_ref[...], (B, H4))

    @pl.loop(0, T)
    def _step(t):
        h_prev0 = h_s[0]
        h_prev1 = h_s[1]

        wide = jnp.dot(h_prev1, kpt,
                       preferred_element_type=jnp.float32)
        wide = wide * diag01
        width = BS
        while width > S:
            width //= 2
            wide = wide[:, :width] + wide[:, width:]
        scores = wide + mask_bias

        m = jnp.max(scores, axis=-1, keepdims=True)
        p = jnp.exp(scores - m)
        w = p / jnp.sum(p, axis=-1, keepdims=True)
        ctx = jnp.sum(w[:, :, None] * enc, axis=1)

        emb = emb_ref[:, pl.ds(t, 1), :][:, 0, :]

        x0 = jnp.concatenate([emb, ctx, h_prev0], axis=-1)
        g0 = jnp.dot(x0, w0, preferred_element_type=jnp.float32) + b0
        sg0 = jax.nn.sigmoid(g0[:, :H3])
        th0 = jnp.tanh(g0[:, H3:])
        c0n = sg0[:, H:H2] * c_s[0] + sg0[:, :H] * th0
        h0n = sg0[:, H2:H3] * jnp.tanh(c0n)

        x1 = jnp.concatenate([h0n, h_prev1], axis=-1)
        g1 = jnp.dot(x1, w1, preferred_element_type=jnp.float32) + b1
        sg1 = jax.nn.sigmoid(g1[:, :H3])
        th1 = jnp.tanh(g1[:, H3:])
        c1n = sg1[:, H:H2] * c_s[1] + sg1[:, :H] * th1
        h1n = sg1[:, H2:H3] * jnp.tanh(c1n)

        h_s[0] = h0n
        c_s[0] = c0n
        h_s[1] = h1n
        c_s[1] = c1n

        rec = jnp.concatenate([h1n, ctx], axis=-1)
        hc_ref[:, pl.ds(t, 1), :] = rec[:, None, :]
        attw_ref[:, pl.ds(t, 1), :] = w[:, None, :]


def _projection_kernel(hc_ref, emb_ref, wo_ref, bo_ref, ws_ref, bs_ref,
                       out_ref):
    x = jnp.concatenate([hc_ref[...], emb_ref[...]],
                        axis=-1).astype(jnp.bfloat16)
    lin = (jnp.dot(x, wo_ref[...], preferred_element_type=jnp.float32)
           + bo_ref[...])
    act = jnp.tanh(lin).astype(jnp.bfloat16)
    out_ref[...] = (jnp.dot(act, ws_ref[...],
                            preferred_element_type=jnp.float32) + bs_ref[...])


def kernel(embedding, wa_t, w0, b0, w1, b1, wo, bo, ws, bs,
           y_tokens, enc_output, mask, h0, c0):
    B, S, I = enc_output.shape
    L, _, H = h0.shape
    V, E = embedding.shape
    inter = bo.shape[-1]
    T = y_tokens.shape[1] - 1
    H2, H3, H4 = 2 * H, 3 * H, 4 * H

    emb_bte = embedding[y_tokens[:, :T]]
    wa = wa_t.T

    def _perm_gates(m):
        return jnp.concatenate([m[:, :H2], m[:, H3:], m[:, H2:H3]], axis=-1)

    w0p, b0p = _perm_gates(w0), _perm_gates(b0)
    w1p, b1p = _perm_gates(w1), _perm_gates(b1)

    hc, attw = pl.pallas_call(
        _recurrent_kernel,
        out_shape=(jax.ShapeDtypeStruct((B, T, H2), jnp.float32),
                   jax.ShapeDtypeStruct((B, T, S), jnp.float32)),
        grid=(1,),
        in_specs=[
            pl.BlockSpec((B, T, E), lambda i: (i, 0, 0)),
            pl.BlockSpec((B, S, I), lambda i: (i, 0, 0)),
            pl.BlockSpec((B, S), lambda i: (i, 0)),
            pl.BlockSpec((L, B, H), lambda i: (0, i, 0)),
            pl.BlockSpec((L, B, H), lambda i: (0, i, 0)),
            pl.BlockSpec((I, H), lambda i: (0, 0)),
            pl.BlockSpec((E + I + H, H4), lambda i: (0, 0)),
            pl.BlockSpec((1, H4), lambda i: (0, 0)),
            pl.BlockSpec((H2, H4), lambda i: (0, 0)),
            pl.BlockSpec((1, H4), lambda i: (0, 0)),
        ],
        out_specs=(pl.BlockSpec((B, T, H2), lambda i: (i, 0, 0)),
                   pl.BlockSpec((B, T, S), lambda i: (i, 0, 0))),
        scratch_shapes=[pltpu.VMEM((L, B, H), jnp.float32),
                        pltpu.VMEM((L, B, H), jnp.float32)],
        compiler_params=pltpu.CompilerParams(
            dimension_semantics=("arbitrary",)),
    )(emb_bte, enc_output, mask, h0, c0, wa, w0p, b0p, w1p, b1p)

    rows = B * T
    n_tiles = 8
    rb = rows // n_tiles
    x_hc = hc.reshape(rows, H2)
    x_emb = emb_bte.reshape(rows, E)
    wo_bf = wo.astype(jnp.bfloat16)
    ws_bf = ws.astype(jnp.bfloat16)

    logits = pl.pallas_call(
        _projection_kernel,
        out_shape=jax.ShapeDtypeStruct((rows, V), jnp.float32),
        grid=(n_tiles,),
        in_specs=[
            pl.BlockSpec((rb, H2), lambda i: (i, 0)),
            pl.BlockSpec((rb, E), lambda i: (i, 0)),
            pl.BlockSpec((H + I + E, inter), lambda i: (0, 0)),
            pl.BlockSpec((1, inter), lambda i: (0, 0)),
            pl.BlockSpec((inter, V), lambda i: (0, 0)),
            pl.BlockSpec((1, V), lambda i: (0, 0)),
        ],
        out_specs=pl.BlockSpec((rb, V), lambda i: (i, 0)),
        compiler_params=pltpu.CompilerParams(
            dimension_semantics=("arbitrary",)),
    )(x_hc, x_emb, wo_bf, bo, ws_bf, bs)

    return logits.reshape(B, T, V), attw


# speedup vs baseline: 1.9900x; 1.1736x over previous
import jax
import jax.numpy as jnp
from jax.experimental import pallas as pl
from jax.experimental.pallas import tpu as pltpu


def _recurrent_kernel(emb_ref, enc_ref, mask_ref, h0_ref, c0_ref, wa_ref,
                      w0_ref, b0_ref, w1_ref, b1_ref,
                      hc_ref, attw_ref, h_s, c_s):
    B, T, E = emb_ref.shape
    _, S, I = enc_ref.shape
    H = h0_ref.shape[2]
    H2, H3, H4 = 2 * H, 3 * H, 4 * H
    BS = B * S

    enc = enc_ref[...]
    enc2d = enc.reshape(BS, I)
    kpt = jax.lax.dot_general(wa_ref[...], enc2d,
                              (((0,), (1,)), ((), ())),
                              preferred_element_type=jnp.float32)

    lane_b = jax.lax.broadcasted_iota(jnp.int32, (B, BS), 1) // S
    row_b = jax.lax.broadcasted_iota(jnp.int32, (B, BS), 0)
    mask_flat = jnp.broadcast_to(mask_ref[...], (B, BS))
    bias_wide = jnp.where((lane_b == row_b) & (mask_flat > 0.0),
                          0.0, -1e30)

    h_s[...] = h0_ref[...]
    c_s[...] = c0_ref[...]

    w0 = w0_ref[...]
    w1 = w1_ref[...]
    b0 = jnp.broadcast_to(b0_ref[...], (B, H4))
    b1 = jnp.broadcast_to(b1_ref[...], (B, H4))

    def one_step(t, state):
        h_prev0, c_prev0, h_prev1, c_prev1 = state

        wide = jnp.dot(h_prev1, kpt,
                       preferred_element_type=jnp.float32) + bias_wide
        pw = jnp.exp(wide)
        ctx_un = jnp.dot(pw, enc2d,
                         preferred_element_type=jnp.float32)

        fold = pw
        width = BS
        while width > S:
            width //= 2
            fold = fold[:, :width] + fold[:, width:]
        r = 1.0 / jnp.sum(fold, axis=-1, keepdims=True)
        w = fold * r
        ctx = ctx_un * r

        emb = emb_ref[:, pl.ds(t, 1), :][:, 0, :]

        x0 = jnp.concatenate([emb, ctx, h_prev0], axis=-1)
        g0 = jnp.dot(x0, w0, preferred_element_type=jnp.float32) + b0
        sif0 = 0.5 * jnp.tanh(0.5 * g0[:, :H2]) + 0.5
        so0 = 0.5 * jnp.tanh(0.5 * g0[:, H3:]) + 0.5
        th0 = jnp.tanh(g0[:, H2:H3])
        c0n = sif0[:, H:] * c_prev0 + sif0[:, :H] * th0
        h0n = so0 * jnp.tanh(c0n)

        x1 = jnp.concatenate([h0n, h_prev1], axis=-1)
        g1 = jnp.dot(x1, w1, preferred_element_type=jnp.float32) + b1
        sif1 = 0.5 * jnp.tanh(0.5 * g1[:, :H2]) + 0.5
        so1 = 0.5 * jnp.tanh(0.5 * g1[:, H3:]) + 0.5
        th1 = jnp.tanh(g1[:, H2:H3])
        c1n = sif1[:, H:] * c_prev1 + sif1[:, :H] * th1
        h1n = so1 * jnp.tanh(c1n)

        rec = jnp.concatenate([h1n, ctx], axis=-1)
        hc_ref[:, pl.ds(t, 1), :] = rec[:, None, :]
        attw_ref[:, pl.ds(t, 1), :] = w[:, None, :]
        return h0n, c0n, h1n, c1n

    UNROLL = 4
    n_pair = T // UNROLL

    @pl.loop(0, n_pair)
    def _pair(tt):
        state = (h_s[0], c_s[0], h_s[1], c_s[1])
        for k in range(UNROLL):
            state = one_step(tt * UNROLL + k, state)
        h_s[0], c_s[0], h_s[1], c_s[1] = state

    if T % UNROLL:
        state = (h_s[0], c_s[0], h_s[1], c_s[1])
        for t in range(n_pair * UNROLL, T):
            state = one_step(t, state)
        h_s[0], c_s[0], h_s[1], c_s[1] = state


def _projection_kernel(hc_ref, emb_ref, wo_ref, bo_ref, ws_ref, bs_ref,
                       out_ref):
    x = jnp.concatenate([hc_ref[...], emb_ref[...]], axis=-1)
    lin = (jnp.dot(x, wo_ref[...], preferred_element_type=jnp.float32)
           + bo_ref[...])
    act = jnp.tanh(lin)
    out_ref[...] = (jnp.dot(act, ws_ref[...],
                            preferred_element_type=jnp.float32) + bs_ref[...])


def kernel(embedding, wa_t, w0, b0, w1, b1, wo, bo, ws, bs,
           y_tokens, enc_output, mask, h0, c0):
    B, S, I = enc_output.shape
    L, _, H = h0.shape
    V, E = embedding.shape
    inter = bo.shape[-1]
    T = y_tokens.shape[1] - 1
    H2, H3, H4 = 2 * H, 3 * H, 4 * H

    emb_bte = embedding[y_tokens[:, :T]]
    wa = wa_t.T

    hc, attw = pl.pallas_call(
        _recurrent_kernel,
        out_shape=(jax.ShapeDtypeStruct((B, T, H2), jnp.float32),
                   jax.ShapeDtypeStruct((B, T, S), jnp.float32)),
        grid=(1,),
        in_specs=[
            pl.BlockSpec((B, T, E), lambda i: (i, 0, 0)),
            pl.BlockSpec((B, S, I), lambda i: (i, 0, 0)),
            pl.BlockSpec((1, B * S), lambda i: (0, 0)),
            pl.BlockSpec((L, B, H), lambda i: (0, i, 0)),
            pl.BlockSpec((L, B, H), lambda i: (0, i, 0)),
            pl.BlockSpec((I, H), lambda i: (0, 0)),
            pl.BlockSpec((E + I + H, H4), lambda i: (0, 0)),
            pl.BlockSpec((1, H4), lambda i: (0, 0)),
            pl.BlockSpec((H2, H4), lambda i: (0, 0)),
            pl.BlockSpec((1, H4), lambda i: (0, 0)),
        ],
        out_specs=(pl.BlockSpec((B, T, H2), lambda i: (i, 0, 0)),
                   pl.BlockSpec((B, T, S), lambda i: (i, 0, 0))),
        scratch_shapes=[pltpu.VMEM((L, B, H), jnp.float32),
                        pltpu.VMEM((L, B, H), jnp.float32)],
        compiler_params=pltpu.CompilerParams(
            dimension_semantics=("arbitrary",)),
    )(emb_bte, enc_output, mask.reshape(1, B * S), h0, c0, wa, w0, b0, w1, b1)

    rows = B * T
    n_tiles = 2
    rb = rows // n_tiles
    x_hc = hc.reshape(rows, H2)
    x_emb = emb_bte.reshape(rows, E)

    logits = pl.pallas_call(
        _projection_kernel,
        out_shape=jax.ShapeDtypeStruct((rows, V), jnp.float32),
        grid=(n_tiles,),
        in_specs=[
            pl.BlockSpec((rb, H2), lambda i: (i, 0)),
            pl.BlockSpec((rb, E), lambda i: (i, 0)),
            pl.BlockSpec((H + I + E, inter), lambda i: (0, 0)),
            pl.BlockSpec((1, inter), lambda i: (0, 0)),
            pl.BlockSpec((inter, V), lambda i: (0, 0)),
            pl.BlockSpec((1, V), lambda i: (0, 0)),
        ],
        out_specs=pl.BlockSpec((rb, V), lambda i: (i, 0)),
        compiler_params=pltpu.CompilerParams(
            dimension_semantics=("arbitrary",)),
    )(x_hc, x_emb, wo, bo, ws, bs)

    return logits.reshape(B, T, V), attw


# speedup vs baseline: 1.9994x; 1.0047x over previous
import jax
import jax.numpy as jnp
from jax.experimental import pallas as pl
from jax.experimental.pallas import tpu as pltpu


def _recurrent_kernel(emb_ref, enc_ref, mask_ref, h0_ref, c0_ref, wa_ref,
                      w0_ref, b0_ref, w1_ref, b1_ref,
                      hc_ref, attw_ref, h_s, c_s):
    B, T, E = emb_ref.shape
    _, S, I = enc_ref.shape
    H = h0_ref.shape[2]
    H2, H3, H4 = 2 * H, 3 * H, 4 * H
    BS = B * S

    enc = enc_ref[...]
    enc2d = enc.reshape(BS, I)
    kpt = jax.lax.dot_general(wa_ref[...], enc2d,
                              (((0,), (1,)), ((), ())),
                              preferred_element_type=jnp.float32)

    lane_b = jax.lax.broadcasted_iota(jnp.int32, (B, BS), 1) // S
    row_b = jax.lax.broadcasted_iota(jnp.int32, (B, BS), 0)
    mask_flat = jnp.broadcast_to(mask_ref[...], (B, BS))
    bias_wide = jnp.where((lane_b == row_b) & (mask_flat > 0.0),
                          0.0, -1e30)

    h_s[...] = h0_ref[...]
    c_s[...] = c0_ref[...]

    w0 = w0_ref[...]
    w1 = w1_ref[...]
    b0 = jnp.broadcast_to(b0_ref[...], (B, H4))
    b1 = jnp.broadcast_to(b1_ref[...], (B, H4))

    def one_step(t, state):
        h_prev0, c_prev0, h_prev1, c_prev1 = state

        wide = jnp.dot(h_prev1, kpt,
                       preferred_element_type=jnp.float32) + bias_wide
        pw = jnp.exp(wide)
        ctx_un = jnp.dot(pw, enc2d,
                         preferred_element_type=jnp.float32)

        fold = pw
        width = BS
        while width > S:
            width //= 2
            fold = fold[:, :width] + fold[:, width:]
        r = 1.0 / jnp.sum(fold, axis=-1, keepdims=True)
        w = fold * r
        ctx = ctx_un * r

        emb = emb_ref[:, pl.ds(t, 1), :][:, 0, :]

        x0 = jnp.concatenate([emb, ctx, h_prev0], axis=-1)
        g0 = jnp.dot(x0, w0, preferred_element_type=jnp.float32) + b0
        sif0 = 0.5 * jnp.tanh(0.5 * g0[:, :H2]) + 0.5
        so0 = 0.5 * jnp.tanh(0.5 * g0[:, H3:]) + 0.5
        th0 = jnp.tanh(g0[:, H2:H3])
        c0n = sif0[:, H:] * c_prev0 + sif0[:, :H] * th0
        h0n = so0 * jnp.tanh(c0n)

        x1 = jnp.concatenate([h0n, h_prev1], axis=-1)
        g1 = jnp.dot(x1, w1, preferred_element_type=jnp.float32) + b1
        sif1 = 0.5 * jnp.tanh(0.5 * g1[:, :H2]) + 0.5
        so1 = 0.5 * jnp.tanh(0.5 * g1[:, H3:]) + 0.5
        th1 = jnp.tanh(g1[:, H2:H3])
        c1n = sif1[:, H:] * c_prev1 + sif1[:, :H] * th1
        h1n = so1 * jnp.tanh(c1n)

        rec = jnp.concatenate([h1n, ctx], axis=-1)
        hc_ref[:, pl.ds(t, 1), :] = rec[:, None, :]
        attw_ref[:, pl.ds(t, 1), :] = w[:, None, :]
        return h0n, c0n, h1n, c1n

    UNROLL = 8
    n_pair = T // UNROLL

    @pl.loop(0, n_pair)
    def _pair(tt):
        state = (h_s[0], c_s[0], h_s[1], c_s[1])
        for k in range(UNROLL):
            state = one_step(tt * UNROLL + k, state)
        h_s[0], c_s[0], h_s[1], c_s[1] = state

    if T % UNROLL:
        state = (h_s[0], c_s[0], h_s[1], c_s[1])
        for t in range(n_pair * UNROLL, T):
            state = one_step(t, state)
        h_s[0], c_s[0], h_s[1], c_s[1] = state


def _projection_kernel(hc_ref, emb_ref, wo_ref, bo_ref, ws_ref, bs_ref,
                       out_ref):
    x = jnp.concatenate([hc_ref[...], emb_ref[...]], axis=-1)
    lin = (jnp.dot(x, wo_ref[...], preferred_element_type=jnp.float32)
           + bo_ref[...])
    act = jnp.tanh(lin)
    out_ref[...] = (jnp.dot(act, ws_ref[...],
                            preferred_element_type=jnp.float32) + bs_ref[...])


def kernel(embedding, wa_t, w0, b0, w1, b1, wo, bo, ws, bs,
           y_tokens, enc_output, mask, h0, c0):
    B, S, I = enc_output.shape
    L, _, H = h0.shape
    V, E = embedding.shape
    inter = bo.shape[-1]
    T = y_tokens.shape[1] - 1
    H2, H3, H4 = 2 * H, 3 * H, 4 * H

    emb_bte = embedding[y_tokens[:, :T]]
    wa = wa_t.T

    hc, attw = pl.pallas_call(
        _recurrent_kernel,
        out_shape=(jax.ShapeDtypeStruct((B, T, H2), jnp.float32),
                   jax.ShapeDtypeStruct((B, T, S), jnp.float32)),
        grid=(1,),
        in_specs=[
            pl.BlockSpec((B, T, E), lambda i: (i, 0, 0)),
            pl.BlockSpec((B, S, I), lambda i: (i, 0, 0)),
            pl.BlockSpec((1, B * S), lambda i: (0, 0)),
            pl.BlockSpec((L, B, H), lambda i: (0, i, 0)),
            pl.BlockSpec((L, B, H), lambda i: (0, i, 0)),
            pl.BlockSpec((I, H), lambda i: (0, 0)),
            pl.BlockSpec((E + I + H, H4), lambda i: (0, 0)),
            pl.BlockSpec((1, H4), lambda i: (0, 0)),
            pl.BlockSpec((H2, H4), lambda i: (0, 0)),
            pl.BlockSpec((1, H4), lambda i: (0, 0)),
        ],
        out_specs=(pl.BlockSpec((B, T, H2), lambda i: (i, 0, 0)),
                   pl.BlockSpec((B, T, S), lambda i: (i, 0, 0))),
        scratch_shapes=[pltpu.VMEM((L, B, H), jnp.float32),
                        pltpu.VMEM((L, B, H), jnp.float32)],
        compiler_params=pltpu.CompilerParams(
            dimension_semantics=("arbitrary",)),
    )(emb_bte, enc_output, mask.reshape(1, B * S), h0, c0, wa, w0, b0, w1, b1)

    rows = B * T
    n_tiles = 2
    rb = rows // n_tiles
    x_hc = hc.reshape(rows, H2)
    x_emb = emb_bte.reshape(rows, E)

    logits = pl.pallas_call(
        _projection_kernel,
        out_shape=jax.ShapeDtypeStruct((rows, V), jnp.float32),
        grid=(n_tiles,),
        in_specs=[
            pl.BlockSpec((rb, H2), lambda i: (i, 0)),
            pl.BlockSpec((rb, E), lambda i: (i, 0)),
            pl.BlockSpec((H + I + E, inter), lambda i: (0, 0)),
            pl.BlockSpec((1, inter), lambda i: (0, 0)),
            pl.BlockSpec((inter, V), lambda i: (0, 0)),
            pl.BlockSpec((1, V), lambda i: (0, 0)),
        ],
        out_specs=pl.BlockSpec((rb, V), lambda i: (i, 0)),
        compiler_params=pltpu.CompilerParams(
            dimension_semantics=("arbitrary",)),
    )(x_hc, x_emb, wo, bo, ws, bs)

    return logits.reshape(B, T, V), attw
